# baseline (speedup 1.0000x reference)
import functools

import jax
import jax.numpy as jnp
from jax import lax
from jax.experimental import pallas as pl
from jax.experimental.pallas import tpu as pltpu
from jax.experimental.pallas import tpu_sc as plsc

VOCAB = 100000
N_CONSTS = 16384
N_ATOMS = 65536
N_PREDS = 64
D_CONST = 128
D_ATOM = 64

NC = 2
NS = 16
NW = NC * NS
LANES = 16
CHUNK = N_ATOMS // NW
BLK = 128
NBLK = CHUNK // BLK


def _sc_interact(x_default, head_pos, tail_pos, constant_table):
    mesh = plsc.VectorSubcoreMesh(core_axis_name="c", subcore_axis_name="s",
                                  num_cores=2)
    hpos3 = head_pos.reshape(NW, NBLK, BLK)
    tpos3 = tail_pos.reshape(NW, NBLK, BLK)

    @functools.partial(
        pl.kernel,
        mesh=mesh,
        out_type=jax.ShapeDtypeStruct((N_ATOMS, D_CONST), jnp.float32),
        scratch_types=[
            pltpu.VMEM((NBLK, BLK), jnp.int32),
            pltpu.VMEM((NBLK, BLK), jnp.int32),
            pltpu.VMEM((NBLK, BLK), jnp.int32),
            pltpu.VMEM((NBLK, BLK), jnp.int32),
            pltpu.VMEM((BLK, D_CONST), jnp.float32),
            pltpu.VMEM((BLK, D_CONST), jnp.float32),
            pltpu.SemaphoreType.DMA,
            pltpu.SemaphoreType.DMA,
        ],
    )
    def k(xdef_hbm, hpos_hbm, tpos_hbm, table_hbm, q_hbm,
          hpos_v, tpos_v, hid_v, tid_v, h_v, t_v, sem, sem_t):
        wid = lax.axis_index("s") * NC + lax.axis_index("c")
        base = pl.multiple_of(wid * CHUNK, CHUNK)
        pltpu.sync_copy(hpos_hbm.at[wid], hpos_v)
        pltpu.sync_copy(tpos_hbm.at[wid], tpos_v)

        trans = []
        for j in range(NBLK):
            trans.append(
                pltpu.async_copy(xdef_hbm.at[hpos_v.at[j]], hid_v.at[j], sem_t))
            trans.append(
                pltpu.async_copy(xdef_hbm.at[tpos_v.at[j]], tid_v.at[j], sem_t))
        for cp in trans:
            cp.wait()

        def blk_body(b, carry):
            cp_h = pltpu.async_copy(table_hbm.at[hid_v.at[b]], h_v, sem)
            cp_t = pltpu.async_copy(table_hbm.at[tid_v.at[b]], t_v, sem)
            cp_h.wait()
            cp_t.wait()

            def mul_row(i, c2):
                for j in range(D_CONST // LANES):
                    s = pl.ds(j * LANES, LANES)
                    h_v[i, s] = h_v[i, s] * t_v[i, s]
                return c2
            lax.fori_loop(0, BLK, mul_row, 0)

            off = pl.multiple_of(base + b * BLK, BLK)
            pltpu.async_copy(h_v, q_hbm.at[pl.ds(off, BLK)], sem).wait()
            return carry

        lax.fori_loop(0, NBLK, blk_body, 0)

    return k(x_default, hpos3, tpos3, constant_table)


BT = 2048


def _tc_project(pred_ids, q, predicate_rel, w_proj, b_proj):
    pred3 = pred_ids.reshape(N_ATOMS // BT, 1, BT)
    b2 = b_proj.reshape(1, D_ATOM)

    def body(pred_ref, q_ref, prel_ref, w_ref, b_ref, out_ref):
        pred = pred_ref[0, 0, :]
        oh = (pred[:, None]
              == lax.broadcasted_iota(jnp.int32, (BT, N_PREDS), 1)
              ).astype(jnp.float32)
        r = jnp.dot(oh, prel_ref[...], preferred_element_type=jnp.float32)
        inter = q_ref[...] * r
        out_ref[...] = jnp.tanh(
            jnp.dot(inter, w_ref[...], preferred_element_type=jnp.float32)
            + b_ref[...])

    return pl.pallas_call(
        body,
        grid=(N_ATOMS // BT,),
        in_specs=[
            pl.BlockSpec((1, 1, BT), lambda i: (i, 0, 0)),
            pl.BlockSpec((BT, D_CONST), lambda i: (i, 0)),
            pl.BlockSpec((N_PREDS, D_CONST), lambda i: (0, 0)),
            pl.BlockSpec((D_CONST, D_ATOM), lambda i: (0, 0)),
            pl.BlockSpec((1, D_ATOM), lambda i: (0, 0)),
        ],
        out_specs=pl.BlockSpec((BT, D_ATOM), lambda i: (i, 0)),
        out_shape=jax.ShapeDtypeStruct((N_ATOMS, D_ATOM), jnp.float32),
    )(pred3, q, predicate_rel, w_proj, b2)


def kernel(X_default, pred_ids, head_pos, tail_pos, constant_table,
           predicate_rel, W_proj, b_proj):
    X_default = X_default.astype(jnp.int32)
    pred_ids = pred_ids.astype(jnp.int32)
    head_pos = head_pos.astype(jnp.int32)
    tail_pos = tail_pos.astype(jnp.int32)
    q = _sc_interact(X_default, head_pos, tail_pos, constant_table)
    return _tc_project(pred_ids, q, predicate_rel, W_proj, b_proj)


# speedup vs baseline: 1.1801x; 1.1801x over previous
import functools

import jax
import jax.numpy as jnp
from jax import lax
from jax.experimental import pallas as pl
from jax.experimental.pallas import tpu as pltpu
from jax.experimental.pallas import tpu_sc as plsc

VOCAB = 100000
N_CONSTS = 16384
N_ATOMS = 65536
N_PREDS = 64
D_CONST = 128
D_ATOM = 64

NC = 2
NS = 16
NW = NC * NS
LANES = 16
CHUNK = N_ATOMS // NW
BLK = 128
NBLK = CHUNK // BLK


def _sc_interact(x_default, head_pos, tail_pos, constant_table):
    mesh = plsc.VectorSubcoreMesh(core_axis_name="c", subcore_axis_name="s",
                                  num_cores=2)
    hpos3 = head_pos.reshape(NW, NBLK, BLK)
    tpos3 = tail_pos.reshape(NW, NBLK, BLK)

    @functools.partial(
        pl.kernel,
        mesh=mesh,
        out_type=jax.ShapeDtypeStruct((N_ATOMS, D_CONST), jnp.float32),
        scratch_types=[
            pltpu.VMEM((NBLK, BLK), jnp.int32),
            pltpu.VMEM((NBLK, BLK), jnp.int32),
            pltpu.VMEM((NBLK, BLK), jnp.int32),
            pltpu.VMEM((NBLK, BLK), jnp.int32),
            pltpu.VMEM((2, BLK, D_CONST), jnp.float32),
            pltpu.VMEM((2, BLK, D_CONST), jnp.float32),
            pltpu.SemaphoreType.DMA,
            pltpu.SemaphoreType.DMA,
            pltpu.SemaphoreType.DMA,
            pltpu.SemaphoreType.DMA,
            pltpu.SemaphoreType.DMA,
        ],
    )
    def k(xdef_hbm, hpos_hbm, tpos_hbm, table_hbm, q_hbm,
          hpos_v, tpos_v, hid_v, tid_v, h_v, t_v,
          sg0, sg1, sw0, sw1, sem_t):
        wid = lax.axis_index("s") * NC + lax.axis_index("c")
        base = pl.multiple_of(wid * CHUNK, CHUNK)
        pltpu.sync_copy(hpos_hbm.at[wid], hpos_v)
        pltpu.sync_copy(tpos_hbm.at[wid], tpos_v)

        trans = []
        for j in range(NBLK):
            trans.append(
                pltpu.async_copy(xdef_hbm.at[hpos_v.at[j]], hid_v.at[j], sem_t))
            trans.append(
                pltpu.async_copy(xdef_hbm.at[tpos_v.at[j]], tid_v.at[j], sem_t))
        for cp in trans:
            cp.wait()

        sg = (sg0, sg1)
        sw = (sw0, sw1)

        def fire_gather(b, s):
            pltpu.async_copy(table_hbm.at[hid_v.at[b]], h_v.at[s], sg[s])
            pltpu.async_copy(table_hbm.at[tid_v.at[b]], t_v.at[s], sg[s])

        def wait_gather(s):
            pltpu.make_async_copy(
                table_hbm.at[hid_v.at[0]], h_v.at[s], sg[s]).wait()
            pltpu.make_async_copy(
                table_hbm.at[tid_v.at[0]], t_v.at[s], sg[s]).wait()

        def wait_write(s):
            pltpu.make_async_copy(
                h_v.at[s], q_hbm.at[pl.ds(0, BLK)], sw[s]).wait()

        fire_gather(0, 0)

        def pair_body(i, carry):
            for s in range(2):
                b = 2 * i + s
                @pl.when(b >= 1)
                def _():
                    wait_write(1 - s)

                @pl.when(b <= NBLK - 2)
                def _():
                    fire_gather(b + 1, 1 - s)

                wait_gather(s)

                def mul_row(r, c2):
                    for j in range(D_CONST // LANES):
                        sl = pl.ds(j * LANES, LANES)
                        h_v[s, r, sl] = h_v[s, r, sl] * t_v[s, r, sl]
                    return c2
                lax.fori_loop(0, BLK, mul_row, 0)

                off = pl.multiple_of(base + b * BLK, BLK)
                pltpu.async_copy(h_v.at[s], q_hbm.at[pl.ds(off, BLK)], sw[s])
            return carry

        lax.fori_loop(0, NBLK // 2, pair_body, 0)
        wait_write(1)

    return k(x_default, hpos3, tpos3, constant_table)


BT = 2048


def _tc_project(pred_ids, q, predicate_rel, w_proj, b_proj):
    pred3 = pred_ids.reshape(N_ATOMS // BT, 1, BT)
    b2 = b_proj.reshape(1, D_ATOM)

    def body(pred_ref, q_ref, prel_ref, w_ref, b_ref, out_ref):
        pred = pred_ref[0, 0, :]
        oh = (pred[:, None]
              == lax.broadcasted_iota(jnp.int32, (BT, N_PREDS), 1)
              ).astype(jnp.float32)
        r = jnp.dot(oh, prel_ref[...], preferred_element_type=jnp.float32)
        inter = q_ref[...] * r
        out_ref[...] = jnp.tanh(
            jnp.dot(inter, w_ref[...], preferred_element_type=jnp.float32)
            + b_ref[...])

    return pl.pallas_call(
        body,
        grid=(N_ATOMS // BT,),
        in_specs=[
            pl.BlockSpec((1, 1, BT), lambda i: (i, 0, 0)),
            pl.BlockSpec((BT, D_CONST), lambda i: (i, 0)),
            pl.BlockSpec((N_PREDS, D_CONST), lambda i: (0, 0)),
            pl.BlockSpec((D_CONST, D_ATOM), lambda i: (0, 0)),
            pl.BlockSpec((1, D_ATOM), lambda i: (0, 0)),
        ],
        out_specs=pl.BlockSpec((BT, D_ATOM), lambda i: (i, 0)),
        out_shape=jax.ShapeDtypeStruct((N_ATOMS, D_ATOM), jnp.float32),
    )(pred3, q, predicate_rel, w_proj, b2)


def kernel(X_default, pred_ids, head_pos, tail_pos, constant_table,
           predicate_rel, W_proj, b_proj):
    X_default = X_default.astype(jnp.int32)
    pred_ids = pred_ids.astype(jnp.int32)
    head_pos = head_pos.astype(jnp.int32)
    tail_pos = tail_pos.astype(jnp.int32)
    q = _sc_interact(X_default, head_pos, tail_pos, constant_table)
    return _tc_project(pred_ids, q, predicate_rel, W_proj, b_proj)


# speedup vs baseline: 1.4640x; 1.2405x over previous
import functools

import jax
import jax.numpy as jnp
from jax import lax
from jax.experimental import pallas as pl
from jax.experimental.pallas import tpu as pltpu
from jax.experimental.pallas import tpu_sc as plsc

VOCAB = 100000
N_CONSTS = 16384
N_ATOMS = 65536
N_PREDS = 64
D_CONST = 128
D_ATOM = 64

NC = 2
NS = 16
NW = NC * NS
LANES = 16
CHUNK = N_ATOMS // NW
BLK = 128
NBLK = CHUNK // BLK


def _sc_interact(x_default, head_pos, tail_pos, constant_table):
    mesh = plsc.VectorSubcoreMesh(core_axis_name="c", subcore_axis_name="s",
                                  num_cores=2)
    hpos3 = head_pos.reshape(NW, NBLK, BLK)
    tpos3 = tail_pos.reshape(NW, NBLK, BLK)

    @functools.partial(
        pl.kernel,
        mesh=mesh,
        out_type=jax.ShapeDtypeStruct((N_ATOMS, D_CONST), jnp.float32),
        scratch_types=[
            pltpu.VMEM((NBLK, BLK), jnp.int32),
            pltpu.VMEM((NBLK, BLK), jnp.int32),
            pltpu.VMEM((NBLK, BLK), jnp.int32),
            pltpu.VMEM((NBLK, BLK), jnp.int32),
            pltpu.VMEM((2, BLK, D_CONST), jnp.float32),
            pltpu.VMEM((2, BLK, D_CONST), jnp.float32),
            pltpu.SemaphoreType.DMA,
            pltpu.SemaphoreType.DMA,
            pltpu.SemaphoreType.DMA,
            pltpu.SemaphoreType.DMA,
            pltpu.SemaphoreType.DMA,
        ],
    )
    def k(xdef_hbm, hpos_hbm, tpos_hbm, table_hbm, q_hbm,
          hpos_v, tpos_v, hid_v, tid_v, h_v, t_v,
          sg0, sg1, sw0, sw1, sem_t):
        wid = lax.axis_index("s") * NC + lax.axis_index("c")
        base = pl.multiple_of(wid * CHUNK, CHUNK)
        pltpu.sync_copy(hpos_hbm.at[wid], hpos_v)
        pltpu.sync_copy(tpos_hbm.at[wid], tpos_v)

        trans = []
        for j in range(NBLK):
            trans.append(
                pltpu.async_copy(xdef_hbm.at[hpos_v.at[j]], hid_v.at[j], sem_t))
            trans.append(
                pltpu.async_copy(xdef_hbm.at[tpos_v.at[j]], tid_v.at[j], sem_t))
        for cp in trans:
            cp.wait()

        sg = (sg0, sg1)
        sw = (sw0, sw1)

        def fire_gather(b, s):
            pltpu.async_copy(table_hbm.at[hid_v.at[b]], h_v.at[s], sg[s])
            pltpu.async_copy(table_hbm.at[tid_v.at[b]], t_v.at[s], sg[s])

        def wait_gather(s):
            pltpu.make_async_copy(
                table_hbm.at[hid_v.at[0]], h_v.at[s], sg[s]).wait()
            pltpu.make_async_copy(
                table_hbm.at[tid_v.at[0]], t_v.at[s], sg[s]).wait()

        def wait_write(s):
            pltpu.make_async_copy(
                h_v.at[s], q_hbm.at[pl.ds(0, BLK)], sw[s]).wait()

        fire_gather(0, 0)

        def pair_body(i, carry):
            for s in range(2):
                b = 2 * i + s
                @pl.when(b >= 1)
                def _():
                    wait_write(1 - s)

                @pl.when(b <= NBLK - 2)
                def _():
                    fire_gather(b + 1, 1 - s)

                wait_gather(s)

                def mul_row(r, c2):
                    for j in range(D_CONST // LANES):
                        sl = pl.ds(j * LANES, LANES)
                        h_v[s, r, sl] = h_v[s, r, sl] * t_v[s, r, sl]
                    return c2
                lax.fori_loop(0, BLK, mul_row, 0)

                off = pl.multiple_of(base + b * BLK, BLK)
                pltpu.async_copy(h_v.at[s], q_hbm.at[pl.ds(off, BLK)], sw[s])
            return carry

        lax.fori_loop(0, NBLK // 2, pair_body, 0)
        wait_write(1)

    return k(x_default, hpos3, tpos3, constant_table)


BT = 2048


def _tc_project(pred_ids, q, predicate_rel, w_proj, b_proj):
    pred3 = pred_ids.reshape(N_ATOMS // BT, 1, BT)
    b2 = b_proj.reshape(D_ATOM, 1)

    def body(pred_ref, q_ref, prel_ref, w_ref, b_ref, out_ref):
        pred = pred_ref[0, 0, :]
        oh = (pred[:, None]
              == lax.broadcasted_iota(jnp.int32, (BT, N_PREDS), 1)
              ).astype(jnp.float32)
        r = jnp.dot(oh, prel_ref[...], preferred_element_type=jnp.float32)
        inter = q_ref[...] * r
        acc = lax.dot_general(w_ref[...], inter, (((0,), (1,)), ((), ())),
                              preferred_element_type=jnp.float32)
        out_ref[...] = jnp.tanh(acc + b_ref[...])

    out_t = pl.pallas_call(
        body,
        grid=(N_ATOMS // BT,),
        in_specs=[
            pl.BlockSpec((1, 1, BT), lambda i: (i, 0, 0)),
            pl.BlockSpec((BT, D_CONST), lambda i: (i, 0)),
            pl.BlockSpec((N_PREDS, D_CONST), lambda i: (0, 0)),
            pl.BlockSpec((D_CONST, D_ATOM), lambda i: (0, 0)),
            pl.BlockSpec((D_ATOM, 1), lambda i: (0, 0)),
        ],
        out_specs=pl.BlockSpec((D_ATOM, BT), lambda i: (0, i)),
        out_shape=jax.ShapeDtypeStruct((D_ATOM, N_ATOMS), jnp.float32),
    )(pred3, q, predicate_rel, w_proj, b2)
    return out_t.T


def kernel(X_default, pred_ids, head_pos, tail_pos, constant_table,
           predicate_rel, W_proj, b_proj):
    X_default = X_default.astype(jnp.int32)
    pred_ids = pred_ids.astype(jnp.int32)
    head_pos = head_pos.astype(jnp.int32)
    tail_pos = tail_pos.astype(jnp.int32)
    q = _sc_interact(X_default, head_pos, tail_pos, constant_table)
    return _tc_project(pred_ids, q, predicate_rel, W_proj, b_proj)


# speedup vs baseline: 1.6093x; 1.0993x over previous
import functools

import jax
import jax.numpy as jnp
from jax import lax
from jax.experimental import pallas as pl
from jax.experimental.pallas import tpu as pltpu
from jax.experimental.pallas import tpu_sc as plsc

VOCAB = 100000
N_CONSTS = 16384
N_ATOMS = 65536
N_PREDS = 64
D_CONST = 128
D_ATOM = 64

NC = 2
NS = 16
NW = NC * NS
LANES = 16
CHUNK = N_ATOMS // NW
BLK = 128
NBLK = CHUNK // BLK


def _sc_interact(x_default, head_pos, tail_pos, constant_table):
    mesh = plsc.VectorSubcoreMesh(core_axis_name="c", subcore_axis_name="s",
                                  num_cores=2)
    hpos3 = head_pos.reshape(NW, NBLK, BLK)
    tpos3 = tail_pos.reshape(NW, NBLK, BLK)

    @functools.partial(
        pl.kernel,
        mesh=mesh,
        out_type=jax.ShapeDtypeStruct((N_ATOMS, D_CONST), jnp.float32),
        scratch_types=[
            pltpu.VMEM((NBLK, BLK), jnp.int32),
            pltpu.VMEM((NBLK, BLK), jnp.int32),
            pltpu.VMEM((NBLK, BLK), jnp.int32),
            pltpu.VMEM((NBLK, BLK), jnp.int32),
            pltpu.VMEM((2, BLK, D_CONST), jnp.float32),
            pltpu.VMEM((2, BLK, D_CONST), jnp.float32),
            pltpu.SemaphoreType.DMA,
            pltpu.SemaphoreType.DMA,
            pltpu.SemaphoreType.DMA,
            pltpu.SemaphoreType.DMA,
            pltpu.SemaphoreType.DMA,
        ],
    )
    def k(xdef_hbm, hpos_hbm, tpos_hbm, table_hbm, q_hbm,
          hpos_v, tpos_v, hid_v, tid_v, h_v, t_v,
          sg0, sg1, sw0, sw1, sem_t):
        wid = lax.axis_index("s") * NC + lax.axis_index("c")
        base = pl.multiple_of(wid * CHUNK, CHUNK)
        pltpu.sync_copy(hpos_hbm.at[wid], hpos_v)
        pltpu.sync_copy(tpos_hbm.at[wid], tpos_v)

        trans = []
        for j in range(NBLK):
            trans.append(
                pltpu.async_copy(xdef_hbm.at[hpos_v.at[j]], hid_v.at[j], sem_t))
            trans.append(
                pltpu.async_copy(xdef_hbm.at[tpos_v.at[j]], tid_v.at[j], sem_t))
        for cp in trans:
            cp.wait()

        sg = (sg0, sg1)
        sw = (sw0, sw1)

        def fire_gather(b, s):
            pltpu.async_copy(table_hbm.at[hid_v.at[b]], h_v.at[s], sg[s])
            pltpu.async_copy(table_hbm.at[tid_v.at[b]], t_v.at[s], sg[s])

        def wait_gather(s):
            pltpu.make_async_copy(
                table_hbm.at[hid_v.at[0]], h_v.at[s], sg[s]).wait()
            pltpu.make_async_copy(
                table_hbm.at[tid_v.at[0]], t_v.at[s], sg[s]).wait()

        def wait_write(s):
            pltpu.make_async_copy(
                h_v.at[s], q_hbm.at[pl.ds(0, BLK)], sw[s]).wait()

        fire_gather(0, 0)

        def pair_body(i, carry):
            for s in range(2):
                b = 2 * i + s
                @pl.when(b >= 1)
                def _():
                    wait_write(1 - s)

                @pl.when(b <= NBLK - 2)
                def _():
                    fire_gather(b + 1, 1 - s)

                wait_gather(s)

                def mul_row(r, c2):
                    for j in range(D_CONST // LANES):
                        sl = pl.ds(j * LANES, LANES)
                        h_v[s, r, sl] = h_v[s, r, sl] * t_v[s, r, sl]
                    return c2
                lax.fori_loop(0, BLK, mul_row, 0)

                off = pl.multiple_of(base + b * BLK, BLK)
                pltpu.async_copy(h_v.at[s], q_hbm.at[pl.ds(off, BLK)], sw[s])
            return carry

        lax.fori_loop(0, NBLK // 2, pair_body, 0)
        wait_write(1)

    return k(x_default, hpos3, tpos3, constant_table)


BT = 4096


def _tc_project(pred_ids, q, predicate_rel, w_proj, b_proj):
    pred3 = pred_ids.reshape(N_ATOMS // BT, 1, BT)
    b2 = b_proj.reshape(D_ATOM, 1)

    def body(pred_ref, q_ref, prel_ref, w_ref, b_ref, out_ref):
        pred = pred_ref[0, 0, :]
        oh = (pred[:, None]
              == lax.broadcasted_iota(jnp.int32, (BT, N_PREDS), 1)
              ).astype(jnp.float32)
        r = jnp.dot(oh, prel_ref[...], preferred_element_type=jnp.float32)
        inter = q_ref[...] * r
        acc = lax.dot_general(w_ref[...], inter, (((0,), (1,)), ((), ())),
                              preferred_element_type=jnp.float32)
        out_ref[...] = jnp.tanh(acc + b_ref[...])

    out_t = pl.pallas_call(
        body,
        grid=(N_ATOMS // BT,),
        in_specs=[
            pl.BlockSpec((1, 1, BT), lambda i: (i, 0, 0)),
            pl.BlockSpec((BT, D_CONST), lambda i: (i, 0)),
            pl.BlockSpec((N_PREDS, D_CONST), lambda i: (0, 0)),
            pl.BlockSpec((D_CONST, D_ATOM), lambda i: (0, 0)),
            pl.BlockSpec((D_ATOM, 1), lambda i: (0, 0)),
        ],
        out_specs=pl.BlockSpec((D_ATOM, BT), lambda i: (0, i)),
        out_shape=jax.ShapeDtypeStruct((D_ATOM, N_ATOMS), jnp.float32),
    )(pred3, q, predicate_rel, w_proj, b2)
    return out_t.T


def kernel(X_default, pred_ids, head_pos, tail_pos, constant_table,
           predicate_rel, W_proj, b_proj):
    X_default = X_default.astype(jnp.int32)
    pred_ids = pred_ids.astype(jnp.int32)
    head_pos = head_pos.astype(jnp.int32)
    tail_pos = tail_pos.astype(jnp.int32)
    q = _sc_interact(X_default, head_pos, tail_pos, constant_table)
    return _tc_project(pred_ids, q, predicate_rel, W_proj, b_proj)


# speedup vs baseline: 1.7025x; 1.0579x over previous
import functools

import jax
import jax.numpy as jnp
from jax import lax
from jax.experimental import pallas as pl
from jax.experimental.pallas import tpu as pltpu
from jax.experimental.pallas import tpu_sc as plsc

VOCAB = 100000
N_CONSTS = 16384
N_ATOMS = 65536
N_PREDS = 64
D_CONST = 128
D_ATOM = 64

NC = 2
NS = 16
NW = NC * NS
LANES = 16
CHUNK = N_ATOMS // NW
BLK = 128
NBLK = CHUNK // BLK


def _sc_interact(x_default, head_pos, tail_pos, constant_table):
    mesh = plsc.VectorSubcoreMesh(core_axis_name="c", subcore_axis_name="s",
                                  num_cores=2)
    hpos3 = head_pos.reshape(NW, NBLK, BLK)
    tpos3 = tail_pos.reshape(NW, NBLK, BLK)

    @functools.partial(
        pl.kernel,
        mesh=mesh,
        out_type=jax.ShapeDtypeStruct((N_ATOMS, D_CONST), jnp.float32),
        scratch_types=[
            pltpu.VMEM((NBLK, BLK), jnp.int32),
            pltpu.VMEM((NBLK, BLK), jnp.int32),
            pltpu.VMEM((NBLK, BLK), jnp.int32),
            pltpu.VMEM((NBLK, BLK), jnp.int32),
            pltpu.VMEM((2, BLK, D_CONST), jnp.float32),
            pltpu.VMEM((2, BLK, D_CONST), jnp.float32),
            pltpu.SemaphoreType.DMA,
            pltpu.SemaphoreType.DMA,
            pltpu.SemaphoreType.DMA,
            pltpu.SemaphoreType.DMA,
            pltpu.SemaphoreType.DMA,
        ],
    )
    def k(xdef_hbm, hpos_hbm, tpos_hbm, table_hbm, q_hbm,
          hpos_v, tpos_v, hid_v, tid_v, h_v, t_v,
          sg0, sg1, sw0, sw1, sem_t):
        wid = lax.axis_index("s") * NC + lax.axis_index("c")
        base = pl.multiple_of(wid * CHUNK, CHUNK)
        pltpu.sync_copy(hpos_hbm.at[wid], hpos_v)
        pltpu.sync_copy(tpos_hbm.at[wid], tpos_v)

        trans = []
        for j in range(NBLK):
            trans.append(
                pltpu.async_copy(xdef_hbm.at[hpos_v.at[j]], hid_v.at[j], sem_t))
            trans.append(
                pltpu.async_copy(xdef_hbm.at[tpos_v.at[j]], tid_v.at[j], sem_t))
        for cp in trans:
            cp.wait()

        sg = (sg0, sg1)
        sw = (sw0, sw1)

        def fire_gather(b, s):
            pltpu.async_copy(table_hbm.at[hid_v.at[b]], h_v.at[s], sg[s])
            pltpu.async_copy(table_hbm.at[tid_v.at[b]], t_v.at[s], sg[s])

        def wait_gather(s):
            pltpu.make_async_copy(
                table_hbm.at[hid_v.at[0]], h_v.at[s], sg[s]).wait()
            pltpu.make_async_copy(
                table_hbm.at[tid_v.at[0]], t_v.at[s], sg[s]).wait()

        def wait_write(s):
            pltpu.make_async_copy(
                h_v.at[s], q_hbm.at[pl.ds(0, BLK)], sw[s]).wait()

        fire_gather(0, 0)

        def pair_body(i, carry):
            for s in range(2):
                b = 2 * i + s
                @pl.when(b >= 1)
                def _():
                    wait_write(1 - s)

                @pl.when(b <= NBLK - 2)
                def _():
                    fire_gather(b + 1, 1 - s)

                wait_gather(s)

                def mul_row(r, c2):
                    for j in range(D_CONST // LANES):
                        sl = pl.ds(j * LANES, LANES)
                        h_v[s, r, sl] = h_v[s, r, sl] * t_v[s, r, sl]
                    return c2
                lax.fori_loop(0, BLK, mul_row, 0)

                off = pl.multiple_of(base + b * BLK, BLK)
                pltpu.async_copy(h_v.at[s], q_hbm.at[pl.ds(off, BLK)], sw[s])
            return carry

        lax.fori_loop(0, NBLK // 2, pair_body, 0)
        wait_write(1)

    return k(x_default, hpos3, tpos3, constant_table)


BT = 8192


def _tc_project(pred_ids, q, predicate_rel, w_proj, b_proj):
    pred3 = pred_ids.reshape(N_ATOMS // BT, 1, BT)
    b2 = b_proj.reshape(D_ATOM, 1)

    def body(pred_ref, q_ref, prel_ref, w_ref, b_ref, out_ref):
        pred = pred_ref[0, 0, :]
        oh = (pred[:, None]
              == lax.broadcasted_iota(jnp.int32, (BT, N_PREDS), 1)
              ).astype(jnp.float32)
        r = jnp.dot(oh, prel_ref[...], preferred_element_type=jnp.float32)
        inter = q_ref[...] * r
        acc = lax.dot_general(w_ref[...], inter, (((0,), (1,)), ((), ())),
                              preferred_element_type=jnp.float32)
        out_ref[...] = jnp.tanh(acc + b_ref[...])

    out_t = pl.pallas_call(
        body,
        grid=(N_ATOMS // BT,),
        in_specs=[
            pl.BlockSpec((1, 1, BT), lambda i: (i, 0, 0)),
            pl.BlockSpec((BT, D_CONST), lambda i: (i, 0)),
            pl.BlockSpec((N_PREDS, D_CONST), lambda i: (0, 0)),
            pl.BlockSpec((D_CONST, D_ATOM), lambda i: (0, 0)),
            pl.BlockSpec((D_ATOM, 1), lambda i: (0, 0)),
        ],
        out_specs=pl.BlockSpec((D_ATOM, BT), lambda i: (0, i)),
        out_shape=jax.ShapeDtypeStruct((D_ATOM, N_ATOMS), jnp.float32),
    )(pred3, q, predicate_rel, w_proj, b2)
    return out_t.T


def kernel(X_default, pred_ids, head_pos, tail_pos, constant_table,
           predicate_rel, W_proj, b_proj):
    X_default = X_default.astype(jnp.int32)
    pred_ids = pred_ids.astype(jnp.int32)
    head_pos = head_pos.astype(jnp.int32)
    tail_pos = tail_pos.astype(jnp.int32)
    q = _sc_interact(X_default, head_pos, tail_pos, constant_table)
    return _tc_project(pred_ids, q, predicate_rel, W_proj, b_proj)


# speedup vs baseline: 1.7244x; 1.0129x over previous
import functools

import jax
import jax.numpy as jnp
from jax import lax
from jax.experimental import pallas as pl
from jax.experimental.pallas import tpu as pltpu
from jax.experimental.pallas import tpu_sc as plsc

VOCAB = 100000
N_CONSTS = 16384
N_ATOMS = 65536
N_PREDS = 64
D_CONST = 128
D_ATOM = 64

NC = 2
NS = 16
NW = NC * NS
LANES = 16
CHUNK = N_ATOMS // NW
BLK = 128
NBLK = CHUNK // BLK


def _sc_interact(x_default, head_pos, tail_pos, constant_table):
    mesh = plsc.VectorSubcoreMesh(core_axis_name="c", subcore_axis_name="s",
                                  num_cores=2)
    hpos3 = head_pos.reshape(NW, NBLK, BLK)
    tpos3 = tail_pos.reshape(NW, NBLK, BLK)

    @functools.partial(
        pl.kernel,
        mesh=mesh,
        out_type=jax.ShapeDtypeStruct((N_ATOMS, D_CONST), jnp.float32),
        scratch_types=[
            pltpu.VMEM((NBLK, BLK), jnp.int32),
            pltpu.VMEM((NBLK, BLK), jnp.int32),
            pltpu.VMEM((NBLK, BLK), jnp.int32),
            pltpu.VMEM((NBLK, BLK), jnp.int32),
            pltpu.VMEM((2, BLK, D_CONST), jnp.float32),
            pltpu.VMEM((2, BLK, D_CONST), jnp.float32),
            pltpu.SemaphoreType.DMA,
            pltpu.SemaphoreType.DMA,
            pltpu.SemaphoreType.DMA,
            pltpu.SemaphoreType.DMA,
            pltpu.SemaphoreType.DMA,
        ],
    )
    def k(xdef_hbm, hpos_hbm, tpos_hbm, table_hbm, q_hbm,
          hpos_v, tpos_v, hid_v, tid_v, h_v, t_v,
          sg0, sg1, sw0, sw1, sem_t):
        wid = lax.axis_index("s") * NC + lax.axis_index("c")
        base = pl.multiple_of(wid * CHUNK, CHUNK)
        pltpu.sync_copy(hpos_hbm.at[wid], hpos_v)
        pltpu.sync_copy(tpos_hbm.at[wid], tpos_v)

        trans = []
        for j in range(NBLK):
            trans.append(
                pltpu.async_copy(xdef_hbm.at[hpos_v.at[j]], hid_v.at[j], sem_t))
            trans.append(
                pltpu.async_copy(xdef_hbm.at[tpos_v.at[j]], tid_v.at[j], sem_t))
        for cp in trans:
            cp.wait()

        sg = (sg0, sg1)
        sw = (sw0, sw1)

        def fire_gather(b, s):
            pltpu.async_copy(table_hbm.at[hid_v.at[b]], h_v.at[s], sg[s])
            pltpu.async_copy(table_hbm.at[tid_v.at[b]], t_v.at[s], sg[s])

        def wait_gather(s):
            pltpu.make_async_copy(
                table_hbm.at[hid_v.at[0]], h_v.at[s], sg[s]).wait()
            pltpu.make_async_copy(
                table_hbm.at[tid_v.at[0]], t_v.at[s], sg[s]).wait()

        def wait_write(s):
            pltpu.make_async_copy(
                h_v.at[s], q_hbm.at[pl.ds(0, BLK)], sw[s]).wait()

        fire_gather(0, 0)

        def pair_body(i, carry):
            for s in range(2):
                b = 2 * i + s
                @pl.when(b >= 1)
                def _():
                    wait_write(1 - s)

                @pl.when(b <= NBLK - 2)
                def _():
                    fire_gather(b + 1, 1 - s)

                wait_gather(s)

                def mul_row(r, c2):
                    for j in range(D_CONST // LANES):
                        sl = pl.ds(j * LANES, LANES)
                        h_v[s, r, sl] = h_v[s, r, sl] * t_v[s, r, sl]
                    return c2
                lax.fori_loop(0, BLK, mul_row, 0)

                off = pl.multiple_of(base + b * BLK, BLK)
                pltpu.async_copy(h_v.at[s], q_hbm.at[pl.ds(off, BLK)], sw[s])
            return carry

        lax.fori_loop(0, NBLK // 2, pair_body, 0)
        wait_write(1)

    return k(x_default, hpos3, tpos3, constant_table)


BT = 16384


def _tc_project(pred_ids, q, predicate_rel, w_proj, b_proj):
    pred3 = pred_ids.reshape(N_ATOMS // BT, 1, BT)
    b2 = b_proj.reshape(D_ATOM, 1)

    def body(pred_ref, q_ref, prel_ref, w_ref, b_ref, out_ref):
        pred = pred_ref[0, 0, :]
        oh = (pred[:, None]
              == lax.broadcasted_iota(jnp.int32, (BT, N_PREDS), 1)
              ).astype(jnp.float32)
        r = jnp.dot(oh, prel_ref[...], preferred_element_type=jnp.float32)
        inter = q_ref[...] * r
        acc = lax.dot_general(w_ref[...], inter, (((0,), (1,)), ((), ())),
                              preferred_element_type=jnp.float32)
        out_ref[...] = jnp.tanh(acc + b_ref[...])

    out_t = pl.pallas_call(
        body,
        grid=(N_ATOMS // BT,),
        in_specs=[
            pl.BlockSpec((1, 1, BT), lambda i: (i, 0, 0)),
            pl.BlockSpec((BT, D_CONST), lambda i: (i, 0)),
            pl.BlockSpec((N_PREDS, D_CONST), lambda i: (0, 0)),
            pl.BlockSpec((D_CONST, D_ATOM), lambda i: (0, 0)),
            pl.BlockSpec((D_ATOM, 1), lambda i: (0, 0)),
        ],
        out_specs=pl.BlockSpec((D_ATOM, BT), lambda i: (0, i)),
        out_shape=jax.ShapeDtypeStruct((D_ATOM, N_ATOMS), jnp.float32),
    )(pred3, q, predicate_rel, w_proj, b2)
    return out_t.T


def kernel(X_default, pred_ids, head_pos, tail_pos, constant_table,
           predicate_rel, W_proj, b_proj):
    X_default = X_default.astype(jnp.int32)
    pred_ids = pred_ids.astype(jnp.int32)
    head_pos = head_pos.astype(jnp.int32)
    tail_pos = tail_pos.astype(jnp.int32)
    q = _sc_interact(X_default, head_pos, tail_pos, constant_table)
    return _tc_project(pred_ids, q, predicate_rel, W_proj, b_proj)
